# baseline (speedup 1.0000x reference)
import functools

import jax
import jax.numpy as jnp
from jax import lax
from jax.experimental import pallas as pl
from jax.experimental.pallas import tpu as pltpu
from jax.experimental.pallas import tpu_sc as plsc

_K_FRAC = 0.25
_MIN32 = -(2 ** 31)
_MAG = 0x7FFFFFFF

_NW = 32
_GRP = 8
_SC_GROUPS = 0
_TC_BLK = 512


def _lane_perm(x, idx):
    dnums = lax.GatherDimensionNumbers(
        offset_dims=(), collapsed_slice_dims=(0,), start_index_map=(0,))
    return lax.gather(x, idx[:, None], dnums, (1,),
                      mode=lax.GatherScatterMode.PROMISE_IN_BOUNDS)


def _lane_sum(x, perms):
    for p in perms:
        x = x + _lane_perm(x, p)
    return x


def _sc_rows_kernel(k, hw, rows_per_w, x_hbm, ts_hbm, tot_hbm,
                    xbuf, kbuf, tsbuf, totbuf):
    wid = lax.axis_index("s") * 2 + lax.axis_index("c")
    base = wid * rows_per_w
    nv = hw // 16
    n_groups = rows_per_w // _GRP
    lanes = 16

    def group_body(g, _):
        row0 = base + g * _GRP
        pltpu.sync_copy(x_hbm.at[pl.ds(row0, _GRP)], xbuf)

        def key_body(j, _):
            for r in range(_GRP):
                v = xbuf[r, pl.ds(j * lanes, lanes)]
                bits = lax.bitcast_convert_type(v, jnp.int32)
                kbuf[r, pl.ds(j * lanes, lanes)] = (
                    bits ^ (lax.shift_right_arithmetic(bits, 31) & _MAG))
            return 0

        lax.fori_loop(0, nv, key_body, 0)

        lane_ids = lax.iota(jnp.int32, lanes)
        perms = [lane_ids ^ s for s in (8, 4, 2, 1)]

        def search_step(i, prefixes):
            shift = 31 - i
            bit = lax.shift_left(jnp.ones((), jnp.int32), shift)
            cands = [p | bit for p in prefixes]
            threshs = [c ^ _MIN32 for c in cands]

            def count_body(j, accs):
                accs = list(accs)
                for u in range(4):
                    for r in range(_GRP):
                        kv = kbuf[r, pl.ds((j * 4 + u) * lanes, lanes)]
                        accs[r] = accs[r] + jnp.where(kv >= threshs[r], 1, 0)
                return tuple(accs)

            zi = jnp.zeros((lanes,), jnp.int32)
            accs = lax.fori_loop(0, nv // 4, count_body, (zi,) * _GRP)
            return tuple(
                jnp.where(_lane_sum(acc, perms) >= k, cand, p)
                for acc, cand, p in zip(accs, cands, prefixes))

        zi = jnp.zeros((lanes,), jnp.int32)
        prefixes = lax.fori_loop(0, 32, search_step, (zi,) * _GRP)

        ts_vec = jnp.zeros((lanes,), jnp.float32)
        tot_vec = jnp.zeros((lanes,), jnp.float32)
        for r in range(_GRP):
            t_key = prefixes[r] ^ _MIN32
            t_val = lax.bitcast_convert_type(
                t_key ^ (lax.shift_right_arithmetic(t_key, 31) & _MAG),
                jnp.float32)

            def tail_body(j, carry, r=r, t_key=t_key):
                cgt, sgt, tot = carry
                for u in range(4):
                    kv = kbuf[r, pl.ds((j * 4 + u) * lanes, lanes)]
                    v = xbuf[r, pl.ds((j * 4 + u) * lanes, lanes)]
                    m = kv > t_key
                    cgt = cgt + jnp.where(m, 1.0, 0.0)
                    sgt = sgt + jnp.where(m, v, 0.0)
                    tot = tot + v
                return cgt, sgt, tot

            zf = jnp.zeros((lanes,), jnp.float32)
            cgt, sgt, tot = lax.fori_loop(
                0, nv // 4, tail_body, (zf, zf, zf))
            cgt_s = _lane_sum(cgt, perms)
            sgt_s = _lane_sum(sgt, perms)
            tot_s = _lane_sum(tot, perms)
            ts_s = sgt_s + (k - cgt_s) * t_val
            ts_vec = jnp.where(lane_ids == r, ts_s, ts_vec)
            tot_vec = jnp.where(lane_ids == r, tot_s, tot_vec)
        tsbuf[pl.ds(g * _GRP, lanes)] = ts_vec
        totbuf[pl.ds(g * _GRP, lanes)] = tot_vec
        return 0

    lax.fori_loop(0, n_groups, group_body, 0)
    pltpu.sync_copy(tsbuf.at[pl.ds(0, rows_per_w)],
                    ts_hbm.at[pl.ds(base, rows_per_w)])
    pltpu.sync_copy(totbuf.at[pl.ds(0, rows_per_w)],
                    tot_hbm.at[pl.ds(base, rows_per_w)])


def _sc_rows(k, hw, sc_rows, x):
    rows_per_w = sc_rows // _NW
    mesh = plsc.VectorSubcoreMesh(core_axis_name="c", subcore_axis_name="s")
    return pl.kernel(
        functools.partial(_sc_rows_kernel, k, hw, rows_per_w),
        mesh=mesh,
        out_type=[
            jax.ShapeDtypeStruct((sc_rows,), jnp.float32),
            jax.ShapeDtypeStruct((sc_rows,), jnp.float32),
        ],
        scratch_types=[
            pltpu.VMEM((_GRP, hw), jnp.float32),
            pltpu.VMEM((_GRP, hw), jnp.int32),
            pltpu.VMEM((rows_per_w + 16,), jnp.float32),
            pltpu.VMEM((rows_per_w + 16,), jnp.float32),
        ],
    )(x)


def _tc_rows_body(k, x_ref, ts_ref, tot_ref, kscratch):
    i = pl.program_id(0)
    cur = jax.lax.rem(i, 2)
    prev = 1 - cur

    x = x_ref[...]
    bits = lax.bitcast_convert_type(x, jnp.int32)
    kscratch[cur] = (bits ^ (lax.shift_right_arithmetic(bits, 31) & _MAG)).T

    @pl.when(i > 0)
    def _():
        key = kscratch[prev]
        r = key.shape[1]

        def step(it, prefix):
            shift = 31 - it
            cand = prefix | lax.shift_left(jnp.ones((), jnp.int32), shift)
            thresh = cand ^ _MIN32
            cnt = jnp.sum((key >= thresh).astype(jnp.int32),
                          axis=0, keepdims=True)
            return jnp.where(cnt >= k, cand, prefix)

        prefix = lax.fori_loop(0, 32, step, jnp.zeros((1, r), jnp.int32))
        t_key = prefix ^ _MIN32
        t_bits = t_key ^ (lax.shift_right_arithmetic(t_key, 31) & _MAG)
        t_val = lax.bitcast_convert_type(t_bits, jnp.float32)

        vals = lax.bitcast_convert_type(
            key ^ (lax.shift_right_arithmetic(key, 31) & _MAG), jnp.float32)
        gt = key > t_key
        cnt_gt = jnp.sum(gt.astype(jnp.int32), axis=0, keepdims=True)
        sum_gt = jnp.sum(jnp.where(gt, vals, 0.0), axis=0, keepdims=True)
        ts_ref[...] = sum_gt + (k - cnt_gt).astype(jnp.float32) * t_val
        tot_ref[...] = jnp.sum(vals, axis=0, keepdims=True)


def _tc_rows(k, hw, sc_rows, x):
    nrows = x.shape[0]
    tc_rows = nrows - sc_rows
    grid = tc_rows // _TC_BLK
    blk0 = sc_rows // _TC_BLK
    last = grid - 1
    ts, tot = pl.pallas_call(
        functools.partial(_tc_rows_body, k),
        grid=(grid + 1,),
        in_specs=[pl.BlockSpec(
            (_TC_BLK, hw), lambda i: (blk0 + jnp.minimum(i, last), 0))],
        out_specs=[
            pl.BlockSpec((1, _TC_BLK), lambda i: (0, jnp.maximum(i - 1, 0))),
            pl.BlockSpec((1, _TC_BLK), lambda i: (0, jnp.maximum(i - 1, 0))),
        ],
        out_shape=[
            jax.ShapeDtypeStruct((1, tc_rows), jnp.float32),
            jax.ShapeDtypeStruct((1, tc_rows), jnp.float32),
        ],
        scratch_shapes=[pltpu.VMEM((2, hw, _TC_BLK), jnp.int32)],
    )(x)
    return ts.reshape(tc_rows), tot.reshape(tc_rows)


def _finalize_body(k, hw, ts_ref, tot_ref, out_ref):
    ts = ts_ref[...]
    tot = tot_ref[...]
    ts_sum = jnp.sum(ts, axis=1, keepdims=True)
    tot_sum = jnp.sum(tot, axis=1, keepdims=True)
    out_ref[...] = ts * (tot_sum / (jnp.float32(hw) * ts_sum))


def kernel(inputs):
    b, c, h, w = inputs.shape
    hw = h * w
    k = int(_K_FRAC * hw)
    nrows = b * c
    x = inputs.reshape(nrows, hw)

    sc_rows = _NW * _GRP * _SC_GROUPS
    parts_ts = []
    parts_tot = []
    if sc_rows > 0:
        ts_sc, tot_sc = _sc_rows(k, hw, sc_rows, x)
        parts_ts.append(ts_sc)
        parts_tot.append(tot_sc)
    if sc_rows < nrows:
        ts_tc, tot_tc = _tc_rows(k, hw, sc_rows, x)
        parts_ts.append(ts_tc)
        parts_tot.append(tot_tc)
    ts = jnp.concatenate(parts_ts) if len(parts_ts) > 1 else parts_ts[0]
    tot = jnp.concatenate(parts_tot) if len(parts_tot) > 1 else parts_tot[0]

    ts = ts.reshape(b, c)
    tot = tot.reshape(b, c)
    out = pl.pallas_call(
        functools.partial(_finalize_body, k, hw),
        out_shape=jax.ShapeDtypeStruct((b, c), jnp.float32),
    )(ts, tot)
    return out


# speedup vs baseline: 1.2253x; 1.2253x over previous
import functools

import jax
import jax.numpy as jnp
from jax import lax
from jax.experimental import pallas as pl
from jax.experimental.pallas import tpu as pltpu
from jax.experimental.pallas import tpu_sc as plsc

_K_FRAC = 0.25
_MIN32 = -(2 ** 31)
_MAG = 0x7FFFFFFF

_NW = 32
_GRP = 8
_SC_GROUPS = 0
_TC_BLK = 512


def _lane_perm(x, idx):
    dnums = lax.GatherDimensionNumbers(
        offset_dims=(), collapsed_slice_dims=(0,), start_index_map=(0,))
    return lax.gather(x, idx[:, None], dnums, (1,),
                      mode=lax.GatherScatterMode.PROMISE_IN_BOUNDS)


def _lane_sum(x, perms):
    for p in perms:
        x = x + _lane_perm(x, p)
    return x


def _sc_rows_kernel(k, hw, rows_per_w, x_hbm, ts_hbm, tot_hbm,
                    xbuf, kbuf, tsbuf, totbuf):
    wid = lax.axis_index("s") * 2 + lax.axis_index("c")
    base = wid * rows_per_w
    nv = hw // 16
    n_groups = rows_per_w // _GRP
    lanes = 16

    def group_body(g, _):
        row0 = base + g * _GRP
        pltpu.sync_copy(x_hbm.at[pl.ds(row0, _GRP)], xbuf)

        def key_body(j, _):
            for r in range(_GRP):
                v = xbuf[r, pl.ds(j * lanes, lanes)]
                bits = lax.bitcast_convert_type(v, jnp.int32)
                kbuf[r, pl.ds(j * lanes, lanes)] = (
                    bits ^ (lax.shift_right_arithmetic(bits, 31) & _MAG))
            return 0

        lax.fori_loop(0, nv, key_body, 0)

        lane_ids = lax.iota(jnp.int32, lanes)
        perms = [lane_ids ^ s for s in (8, 4, 2, 1)]

        def search_step(i, prefixes):
            shift = 31 - i
            bit = lax.shift_left(jnp.ones((), jnp.int32), shift)
            cands = [p | bit for p in prefixes]
            threshs = [c ^ _MIN32 for c in cands]

            def count_body(j, accs):
                accs = list(accs)
                for u in range(4):
                    for r in range(_GRP):
                        kv = kbuf[r, pl.ds((j * 4 + u) * lanes, lanes)]
                        accs[r] = accs[r] + jnp.where(kv >= threshs[r], 1, 0)
                return tuple(accs)

            zi = jnp.zeros((lanes,), jnp.int32)
            accs = lax.fori_loop(0, nv // 4, count_body, (zi,) * _GRP)
            return tuple(
                jnp.where(_lane_sum(acc, perms) >= k, cand, p)
                for acc, cand, p in zip(accs, cands, prefixes))

        zi = jnp.zeros((lanes,), jnp.int32)
        prefixes = lax.fori_loop(0, 32, search_step, (zi,) * _GRP)

        ts_vec = jnp.zeros((lanes,), jnp.float32)
        tot_vec = jnp.zeros((lanes,), jnp.float32)
        for r in range(_GRP):
            t_key = prefixes[r] ^ _MIN32
            t_val = lax.bitcast_convert_type(
                t_key ^ (lax.shift_right_arithmetic(t_key, 31) & _MAG),
                jnp.float32)

            def tail_body(j, carry, r=r, t_key=t_key):
                cgt, sgt, tot = carry
                for u in range(4):
                    kv = kbuf[r, pl.ds((j * 4 + u) * lanes, lanes)]
                    v = xbuf[r, pl.ds((j * 4 + u) * lanes, lanes)]
                    m = kv > t_key
                    cgt = cgt + jnp.where(m, 1.0, 0.0)
                    sgt = sgt + jnp.where(m, v, 0.0)
                    tot = tot + v
                return cgt, sgt, tot

            zf = jnp.zeros((lanes,), jnp.float32)
            cgt, sgt, tot = lax.fori_loop(
                0, nv // 4, tail_body, (zf, zf, zf))
            cgt_s = _lane_sum(cgt, perms)
            sgt_s = _lane_sum(sgt, perms)
            tot_s = _lane_sum(tot, perms)
            ts_s = sgt_s + (k - cgt_s) * t_val
            ts_vec = jnp.where(lane_ids == r, ts_s, ts_vec)
            tot_vec = jnp.where(lane_ids == r, tot_s, tot_vec)
        tsbuf[pl.ds(g * _GRP, lanes)] = ts_vec
        totbuf[pl.ds(g * _GRP, lanes)] = tot_vec
        return 0

    lax.fori_loop(0, n_groups, group_body, 0)
    pltpu.sync_copy(tsbuf.at[pl.ds(0, rows_per_w)],
                    ts_hbm.at[pl.ds(base, rows_per_w)])
    pltpu.sync_copy(totbuf.at[pl.ds(0, rows_per_w)],
                    tot_hbm.at[pl.ds(base, rows_per_w)])


def _sc_rows(k, hw, sc_rows, x):
    rows_per_w = sc_rows // _NW
    mesh = plsc.VectorSubcoreMesh(core_axis_name="c", subcore_axis_name="s")
    return pl.kernel(
        functools.partial(_sc_rows_kernel, k, hw, rows_per_w),
        mesh=mesh,
        out_type=[
            jax.ShapeDtypeStruct((sc_rows,), jnp.float32),
            jax.ShapeDtypeStruct((sc_rows,), jnp.float32),
        ],
        scratch_types=[
            pltpu.VMEM((_GRP, hw), jnp.float32),
            pltpu.VMEM((_GRP, hw), jnp.int32),
            pltpu.VMEM((rows_per_w + 16,), jnp.float32),
            pltpu.VMEM((rows_per_w + 16,), jnp.float32),
        ],
    )(x)


def _tc_rows_body(k, xt_ref, ts_ref, tot_ref):
    x = xt_ref[...]
    bits = lax.bitcast_convert_type(x, jnp.int32)
    key = bits ^ (lax.shift_right_arithmetic(bits, 31) & _MAG)
    r = x.shape[1]

    prefix = jnp.zeros((1, r), jnp.int32)
    for b in range(31, -1, -1):
        bit = -(2 ** 31) if b == 31 else (1 << b)
        cand = prefix | bit
        thresh = cand ^ _MIN32
        cnt = jnp.sum((key >= thresh).astype(jnp.int32), axis=0, keepdims=True)
        prefix = jnp.where(cnt >= k, cand, prefix)

    t_key = prefix ^ _MIN32
    t_bits = t_key ^ (lax.shift_right_arithmetic(t_key, 31) & _MAG)
    t_val = lax.bitcast_convert_type(t_bits, jnp.float32)

    gt = key > t_key
    cnt_gt = jnp.sum(gt.astype(jnp.int32), axis=0, keepdims=True)
    sum_gt = jnp.sum(jnp.where(gt, x, 0.0), axis=0, keepdims=True)
    ts_ref[...] = sum_gt + (k - cnt_gt).astype(jnp.float32) * t_val
    tot_ref[...] = jnp.sum(x, axis=0, keepdims=True)


def _tc_rows(k, hw, sc_rows, xt):
    nrows = xt.shape[1]
    tc_rows = nrows - sc_rows
    grid = tc_rows // _TC_BLK
    blk0 = sc_rows // _TC_BLK
    ts, tot = pl.pallas_call(
        functools.partial(_tc_rows_body, k),
        grid=(grid,),
        in_specs=[pl.BlockSpec((hw, _TC_BLK), lambda i: (0, blk0 + i))],
        out_specs=[
            pl.BlockSpec((1, _TC_BLK), lambda i: (0, i)),
            pl.BlockSpec((1, _TC_BLK), lambda i: (0, i)),
        ],
        out_shape=[
            jax.ShapeDtypeStruct((1, tc_rows), jnp.float32),
            jax.ShapeDtypeStruct((1, tc_rows), jnp.float32),
        ],
    )(xt)
    return ts.reshape(tc_rows), tot.reshape(tc_rows)


def _finalize_body(k, hw, ts_ref, tot_ref, out_ref):
    ts = ts_ref[...]
    tot = tot_ref[...]
    ts_sum = jnp.sum(ts, axis=1, keepdims=True)
    tot_sum = jnp.sum(tot, axis=1, keepdims=True)
    out_ref[...] = ts * (tot_sum / (jnp.float32(hw) * ts_sum))


def kernel(inputs):
    b, c, h, w = inputs.shape
    hw = h * w
    k = int(_K_FRAC * hw)
    nrows = b * c
    x = inputs.reshape(nrows, hw)

    sc_rows = _NW * _GRP * _SC_GROUPS
    parts_ts = []
    parts_tot = []
    if sc_rows > 0:
        ts_sc, tot_sc = _sc_rows(k, hw, sc_rows, x)
        parts_ts.append(ts_sc)
        parts_tot.append(tot_sc)
    if sc_rows < nrows:
        xt = x[sc_rows:].T
        ts_tc, tot_tc = _tc_rows(k, hw, 0, xt)
        parts_ts.append(ts_tc)
        parts_tot.append(tot_tc)
    ts = jnp.concatenate(parts_ts) if len(parts_ts) > 1 else parts_ts[0]
    tot = jnp.concatenate(parts_tot) if len(parts_tot) > 1 else parts_tot[0]

    ts = ts.reshape(b, c)
    tot = tot.reshape(b, c)
    out = pl.pallas_call(
        functools.partial(_finalize_body, k, hw),
        out_shape=jax.ShapeDtypeStruct((b, c), jnp.float32),
    )(ts, tot)
    return out
